# baseline (speedup 1.0000x reference)
import functools

import jax
import jax.numpy as jnp
from jax import lax
from jax.experimental import pallas as pl
from jax.experimental.pallas import tpu as pltpu
from jax.experimental.pallas import tpu_sc as plsc

E = 8
D = 2048
FF = 2048
S = 2048

NA = S * 2
TB = 128
NB = NA // TB
MAX_PAIRS = NB + E
RT = 256
FB1 = 1024
NF1 = FF // FB1

NW = 32

K2_RPW = NA // NW
K2_CH = 16
K2_NCH = K2_RPW // K2_CH

TPW = S // NW
K4_CH = 4
K4_NCH = TPW // K4_CH


def _router_body(x_ref, wr_ref, logits_ref, idx_ref, wts_ref):
    x = x_ref[...]
    wr = wr_ref[...]
    logits = jnp.dot(x, wr, preferred_element_type=jnp.float32)
    logits_ref[...] = logits
    lane = lax.broadcasted_iota(jnp.int32, logits.shape, 1)
    m1 = jnp.max(logits, axis=1, keepdims=True)
    i1 = jnp.min(jnp.where(logits == m1, lane, E), axis=1, keepdims=True)
    masked = jnp.where(lane == i1, -jnp.inf, logits)
    m2 = jnp.max(masked, axis=1, keepdims=True)
    i2 = jnp.min(jnp.where(masked == m2, lane, E), axis=1, keepdims=True)
    w1 = 1.0 / (1.0 + jnp.exp(m2 - m1))
    idx_ref[...] = jnp.concatenate([i1, i2], axis=1)
    wts_ref[...] = jnp.concatenate([w1, 1.0 - w1], axis=1)


def _run_router(flat, Wr):
    return pl.pallas_call(
        _router_body,
        grid=(S // RT,),
        in_specs=[
            pl.BlockSpec((RT, D), lambda i: (i, 0)),
            pl.BlockSpec((D, E), lambda i: (0, 0)),
        ],
        out_specs=[
            pl.BlockSpec((RT, E), lambda i: (i, 0)),
            pl.BlockSpec((RT, 2), lambda i: (i, 0)),
            pl.BlockSpec((RT, 2), lambda i: (i, 0)),
        ],
        out_shape=[
            jax.ShapeDtypeStruct((S, E), jnp.float32),
            jax.ShapeDtypeStruct((S, 2), jnp.int32),
            jax.ShapeDtypeStruct((S, 2), jnp.float32),
        ],
    )(flat, Wr)


def _routing_metadata(topi):
    i32 = jnp.int32
    e_all = jnp.concatenate([topi[:, 0], topi[:, 1]])
    onehot = (e_all[:, None] == jnp.arange(E, dtype=i32)[None, :]).astype(i32)
    csum = jnp.cumsum(onehot, axis=0)
    counts = csum[-1]
    off = jnp.concatenate([jnp.zeros(1, i32), jnp.cumsum(counts)])
    rank = off[e_all] + jnp.sum(onehot * csum, axis=1) - 1
    gat = jnp.stack([rank[:S], rank[S:]], axis=1).reshape(-1)

    blo = off[:-1] // TB
    bhi = (off[1:] - 1) // TB
    nb_e = jnp.where(counts > 0, bhi - blo + 1, 0)
    poff = jnp.concatenate([jnp.zeros(1, i32), jnp.cumsum(nb_e)])
    p_ar = jnp.arange(MAX_PAIRS, dtype=i32)
    e_p = jnp.sum((p_ar[:, None] >= poff[None, 1:]).astype(i32), axis=1)
    active = p_ar < poff[-1]
    e_pc = jnp.minimum(e_p, E - 1)
    e_last = jnp.max(jnp.where(counts > 0, jnp.arange(E, dtype=i32), 0))
    b_p = blo[e_pc] + (p_ar - poff[e_pc])
    b_p = jnp.where(active, b_p, NB - 1)
    e_m = jnp.where(active, e_pc, e_last)
    row_lo = jnp.where(active, jnp.maximum(off[e_pc], b_p * TB), 0)
    row_hi = jnp.where(active, jnp.minimum(off[e_pc + 1], (b_p + 1) * TB), 0)
    meta = jnp.stack([b_p, e_m, row_lo, row_hi]).astype(i32)
    return rank, gat, meta


def _sc_permute_body(flat_hbm, rk_hbm, out_hbm, idx_v, bufs, rsem, wsem):
    wid = lax.axis_index("s") * 2 + lax.axis_index("c")
    pltpu.sync_copy(rk_hbm.at[wid], idx_v)
    tokbase = (wid % (S // K2_RPW)) * K2_RPW
    rd = [None, None]
    wr = [None, None]
    rd[0] = pltpu.async_copy(
        flat_hbm.at[pl.ds(tokbase, K2_CH)], bufs.at[0], rsem)
    for c in range(K2_NCH):
        cur = c % 2
        nb = (c + 1) % 2
        if c + 1 < K2_NCH:
            if wr[nb] is not None:
                wr[nb].wait()
                wr[nb] = None
            rd[nb] = pltpu.async_copy(
                flat_hbm.at[pl.ds(tokbase + (c + 1) * K2_CH, K2_CH)],
                bufs.at[nb], rsem)
        rd[cur].wait()
        wr[cur] = pltpu.async_copy(bufs.at[cur], out_hbm.at[idx_v.at[c]], wsem)
    for z in range(2):
        if wr[z] is not None:
            wr[z].wait()


def _run_sc_permute(flat, rank):
    mesh = plsc.VectorSubcoreMesh(core_axis_name="c", subcore_axis_name="s")
    k = pl.kernel(
        _sc_permute_body,
        out_type=jax.ShapeDtypeStruct((NA, D), jnp.float32),
        mesh=mesh,
        scratch_types=[
            pltpu.VMEM((K2_NCH, K2_CH), jnp.int32),
            pltpu.VMEM((2, K2_CH, D), jnp.float32),
            pltpu.SemaphoreType.DMA,
            pltpu.SemaphoreType.DMA,
        ],
    )
    return k(flat, rank.reshape(NW, K2_NCH, K2_CH))


def _group_h_body(m_ref, x_ref, w1_ref, w3_ref, h_ref):
    p = pl.program_id(1)
    b = m_ref[0, p]
    lo = m_ref[2, p]
    hi = m_ref[3, p]
    x = x_ref[...]
    a = jnp.dot(x, w1_ref[0], preferred_element_type=jnp.float32)
    g = jnp.dot(x, w3_ref[0], preferred_element_type=jnp.float32)
    h = a * jax.nn.sigmoid(a) * g
    rows = b * TB + lax.broadcasted_iota(jnp.int32, (TB, 1), 0)
    mask = ((rows >= lo) & (rows < hi)).astype(jnp.float32)
    contrib = (mask * h).astype(jnp.bfloat16)
    prev_b = m_ref[0, jnp.maximum(p - 1, 0)]
    first = (p == 0) | (b != prev_b)

    @pl.when(first)
    def _():
        h_ref[...] = contrib

    @pl.when(jnp.logical_not(first))
    def _():
        h_ref[...] = h_ref[...] + contrib


def _group_y_body(m_ref, h_ref, w2_ref, o_ref):
    p = pl.program_id(0)
    b = m_ref[0, p]
    lo = m_ref[2, p]
    hi = m_ref[3, p]
    y = jnp.dot(h_ref[...].astype(jnp.float32), w2_ref[0],
                preferred_element_type=jnp.float32)
    rows = b * TB + lax.broadcasted_iota(jnp.int32, (TB, 1), 0)
    mask = ((rows >= lo) & (rows < hi)).astype(jnp.float32)
    contrib = mask * y
    prev_b = m_ref[0, jnp.maximum(p - 1, 0)]
    first = (p == 0) | (b != prev_b)

    @pl.when(first)
    def _():
        o_ref[...] = contrib

    @pl.when(jnp.logical_not(first))
    def _():
        o_ref[...] = o_ref[...] + contrib


def _run_grouped(x_s, W1, W3, W2, meta):
    h_spec = pltpu.PrefetchScalarGridSpec(
        num_scalar_prefetch=1,
        grid=(NF1, MAX_PAIRS),
        in_specs=[
            pl.BlockSpec((TB, D), lambda f, p, m: (m[0, p], 0)),
            pl.BlockSpec((1, D, FB1), lambda f, p, m: (m[1, p], 0, f)),
            pl.BlockSpec((1, D, FB1), lambda f, p, m: (m[1, p], 0, f)),
        ],
        out_specs=pl.BlockSpec((TB, FB1), lambda f, p, m: (m[0, p], f)),
    )
    h_s = pl.pallas_call(
        _group_h_body,
        grid_spec=h_spec,
        out_shape=jax.ShapeDtypeStruct((NA, FF), jnp.bfloat16),
        compiler_params=pltpu.CompilerParams(
            dimension_semantics=("arbitrary", "arbitrary")),
    )(meta, x_s, W1, W3)
    y_spec = pltpu.PrefetchScalarGridSpec(
        num_scalar_prefetch=1,
        grid=(MAX_PAIRS,),
        in_specs=[
            pl.BlockSpec((TB, FF), lambda p, m: (m[0, p], 0)),
            pl.BlockSpec((1, FF, D), lambda p, m: (m[1, p], 0, 0)),
        ],
        out_specs=pl.BlockSpec((TB, D), lambda p, m: (m[0, p], 0)),
    )
    return pl.pallas_call(
        _group_y_body,
        grid_spec=y_spec,
        out_shape=jax.ShapeDtypeStruct((NA, D), jnp.float32),
        compiler_params=pltpu.CompilerParams(
            dimension_semantics=("arbitrary",)),
    )(meta, h_s, W2)


def _shared_h_body(x_ref, w1_ref, w3_ref, h_ref):
    x = x_ref[...]
    a = jnp.dot(x, w1_ref[...], preferred_element_type=jnp.float32)
    g = jnp.dot(x, w3_ref[...], preferred_element_type=jnp.float32)
    h_ref[...] = (a * jax.nn.sigmoid(a) * g).astype(jnp.bfloat16)


def _shared_y_body(h_ref, w2_ref, o_ref):
    o_ref[...] = jnp.dot(h_ref[...].astype(jnp.float32), w2_ref[...],
                         preferred_element_type=jnp.float32)


def _run_shared(flat, Ws1, Ws3, Ws2):
    h_sh = pl.pallas_call(
        _shared_h_body,
        grid=(NF1, S // TB),
        in_specs=[
            pl.BlockSpec((TB, D), lambda f, t: (t, 0)),
            pl.BlockSpec((D, FB1), lambda f, t: (0, f)),
            pl.BlockSpec((D, FB1), lambda f, t: (0, f)),
        ],
        out_specs=pl.BlockSpec((TB, FB1), lambda f, t: (t, f)),
        out_shape=jax.ShapeDtypeStruct((S, FF), jnp.bfloat16),
        compiler_params=pltpu.CompilerParams(
            dimension_semantics=("arbitrary", "arbitrary")),
    )(flat, Ws1, Ws3)
    return pl.pallas_call(
        _shared_y_body,
        grid=(S // TB,),
        in_specs=[
            pl.BlockSpec((TB, FF), lambda t: (t, 0)),
            pl.BlockSpec((FF, D), lambda t: (0, 0)),
        ],
        out_specs=pl.BlockSpec((TB, D), lambda t: (t, 0)),
        out_shape=jax.ShapeDtypeStruct((S, D), jnp.float32),
        compiler_params=pltpu.CompilerParams(
            dimension_semantics=("arbitrary",)),
    )(h_sh, Ws2)


def _sc_combine_body(ys_hbm, sh_hbm, gat_hbm, tw_hbm, out_hbm,
                     idx_v, rbuf, sbuf, wbuf, obuf, gsem, ssem, wsem, osem):
    wid = lax.axis_index("s") * 2 + lax.axis_index("c")
    pltpu.sync_copy(gat_hbm.at[wid], idx_v)
    tokbase = wid * TPW
    cps = {}
    ocp = [None, None]

    def start(c):
        nb = c % 2
        cps[c] = (
            pltpu.async_copy(ys_hbm.at[idx_v.at[c]], rbuf.at[nb], gsem),
            pltpu.async_copy(
                sh_hbm.at[pl.ds(tokbase + c * K4_CH, K4_CH)], sbuf.at[nb], ssem),
            pltpu.async_copy(
                tw_hbm.at[pl.ds(tokbase + c * K4_CH, K4_CH)], wbuf.at[nb], wsem),
        )

    start(0)
    for c in range(K4_NCH):
        cur = c % 2
        if c + 1 < K4_NCH:
            if ocp[(c + 1) % 2] is not None:
                ocp[(c + 1) % 2].wait()
                ocp[(c + 1) % 2] = None
            start(c + 1)
        for cp in cps.pop(c):
            cp.wait()
        if ocp[cur] is not None:
            ocp[cur].wait()
            ocp[cur] = None
        def body(j, carry):
            sl = pl.ds(j * 16, 16)
            for t in range(K4_CH):
                w0v = wbuf[cur, t, 0, :]
                w1v = wbuf[cur, t, 1, :]
                obuf[cur, t, sl] = (sbuf[cur, t, sl]
                                    + w0v * rbuf[cur, 2 * t, sl]
                                    + w1v * rbuf[cur, 2 * t + 1, sl])
            return carry

        lax.fori_loop(0, D // 16, body, 0)
        ocp[cur] = pltpu.async_copy(
            obuf.at[cur], out_hbm.at[pl.ds(tokbase + c * K4_CH, K4_CH)], osem)
    for z in range(2):
        if ocp[z] is not None:
            ocp[z].wait()


def _run_sc_combine(y_s, shared_out, gat, topw):
    mesh = plsc.VectorSubcoreMesh(core_axis_name="c", subcore_axis_name="s")
    k = pl.kernel(
        _sc_combine_body,
        out_type=jax.ShapeDtypeStruct((S, D), jnp.float32),
        mesh=mesh,
        scratch_types=[
            pltpu.VMEM((K4_NCH, 2 * K4_CH), jnp.int32),
            pltpu.VMEM((2, 2 * K4_CH, D), jnp.float32),
            pltpu.VMEM((2, K4_CH, D), jnp.float32),
            pltpu.VMEM((2, K4_CH, 2, 16), jnp.float32),
            pltpu.VMEM((2, K4_CH, D), jnp.float32),
            pltpu.SemaphoreType.DMA,
            pltpu.SemaphoreType.DMA,
            pltpu.SemaphoreType.DMA,
            pltpu.SemaphoreType.DMA,
        ],
    )
    topw16 = jnp.broadcast_to(topw[:, :, None], (S, 2, 16))
    return k(y_s, shared_out, gat.reshape(NW, K4_NCH, 2 * K4_CH), topw16)


def kernel(hidden_states, W1, W2, W3, Ws1, Ws2, Ws3, Wr):
    b, s, d = hidden_states.shape
    flat = hidden_states.reshape(-1, d)
    logits, topi, topw = _run_router(flat, Wr)
    rank, gat, meta = _routing_metadata(topi)
    x_s = _run_sc_permute(flat, rank)
    y_s = _run_grouped(x_s, W1, W3, W2, meta)
    shared_out = _run_shared(flat, Ws1, Ws3, Ws2)
    final = _run_sc_combine(y_s, shared_out, gat, topw)
    return final.reshape(b, s, d), logits
